# baseline (speedup 1.0000x reference)
import functools

import jax
import jax.numpy as jnp
from jax import lax
from jax.experimental import pallas as pl
from jax.experimental.pallas import tpu as pltpu
from jax.experimental.pallas import tpu_sc as plsc

VOCAB = 1000000
EMBED_DIM = 32
BATCH = 4096
HIST = 50

NC, NS = 2, 16
NW = NC * NS
TOTAL = BATCH * HIST
B_PER_W = TOTAL // NW
CHUNK = 1600
NCHUNK = B_PER_W // CHUNK

LANES = 128
VTILES = (VOCAB + LANES - 1) // LANES
VPAD = VTILES * LANES
TBL_WORDS = VPAD * EMBED_DIM
BT = BATCH // LANES

_mesh = plsc.VectorSubcoreMesh(core_axis_name="c", subcore_axis_name="s")


@functools.partial(
    pl.kernel,
    mesh=_mesh,
    out_type=(
        jax.ShapeDtypeStruct((TBL_WORDS,), jnp.float32),
        jax.ShapeDtypeStruct((TOTAL,), jnp.int32),
    ),
    scratch_types=[
        pltpu.VMEM((32, LANES), jnp.float32),
        pltpu.VMEM((4096,), jnp.float32),
        pltpu.VMEM((8, LANES), jnp.int32),
        pltpu.VMEM((B_PER_W,), jnp.int32),
        pltpu.SemaphoreType.DMA,
        pltpu.SemaphoreType.DMA,
    ],
    compiler_params=pltpu.CompilerParams(
        use_tc_tiling_on_sc=True, needs_layout_passes=False),
)
def _reformat_kernel(tab_hbm, idx_hbm, tbl_out, idx_out, stg, blk, istg, iblk,
                     sem_in, sem_out):
    wid = lax.axis_index("s") * NC + lax.axis_index("c")
    iota = lax.broadcasted_iota(jnp.int32, (16,), 0)

    ilane = iota * HIST
    for k in range(7):
        hstart = 8 * k
        nrows = min(8, HIST - hstart)
        pltpu.async_copy(
            idx_hbm.at[pl.ds(hstart, nrows), pl.ds(wid * LANES, LANES)],
            istg.at[pl.ds(0, nrows)], sem_in).wait()
        for r in range(nrows):
            h = hstart + r
            vs = [istg[r, pl.ds(g * 16, 16)] for g in range(8)]
            for g in range(8):
                plsc.store_scatter(iblk, [ilane + (g * 16 * HIST + h)], vs[g])
    pltpu.async_copy(iblk, idx_out.at[pl.ds(wid * B_PER_W, B_PER_W)],
                     sem_out).wait()

    vlane = iota * EMBED_DIM

    def body(j, _):
        ct = j * NW + wid

        @pl.when(ct < VTILES)
        def _():
            off = pl.multiple_of(ct * LANES, LANES)
            pltpu.async_copy(tab_hbm.at[:, pl.ds(off, LANES)], stg,
                             sem_in).wait()
            for e in range(EMBED_DIM):
                vs = [stg[e, pl.ds(g * 16, 16)] for g in range(8)]
                for g in range(8):
                    plsc.store_scatter(
                        blk, [vlane + (g * 16 * EMBED_DIM + e)], vs[g])
            pltpu.async_copy(
                blk, tbl_out.at[pl.ds(ct * (LANES * EMBED_DIM),
                                      LANES * EMBED_DIM)],
                sem_out).wait()
        return _

    lax.fori_loop(0, (VTILES + NW - 1) // NW, body, None)


@functools.partial(
    pl.kernel,
    mesh=_mesh,
    out_type=jax.ShapeDtypeStruct((BATCH, HIST, EMBED_DIM), jnp.float32),
    scratch_types=[
        pltpu.VMEM((CHUNK,), jnp.int32),
        pltpu.VMEM((CHUNK, EMBED_DIM), jnp.float32),
        pltpu.SemaphoreType.DMA,
    ],
    compiler_params=pltpu.CompilerParams(use_tc_tiling_on_sc=False),
)
def _gather_kernel(table_hbm, idx_hbm, out_hbm, idx_v, rows_v, sem):
    wid = lax.axis_index("s") * NC + lax.axis_index("c")
    base = wid * B_PER_W
    bstart = wid * (BATCH // NW)
    for c in range(NCHUNK):
        off = base + c * CHUNK
        pltpu.sync_copy(idx_hbm.at[pl.ds(off, CHUNK)], idx_v)
        pltpu.async_copy(table_hbm.at[idx_v], rows_v, sem).wait()
        for b in range(CHUNK // HIST):
            pltpu.sync_copy(rows_v.at[pl.ds(b * HIST, HIST)],
                            out_hbm.at[bstart + c * (CHUNK // HIST) + b])


def kernel(inputs, embeddings):
    tbl_lin, idx_lin = _reformat_kernel(embeddings.T, inputs.T)
    table = jnp.reshape(tbl_lin, (VPAD, EMBED_DIM))
    return _gather_kernel(table, idx_lin)


# speedup vs baseline: 1.2778x; 1.2778x over previous
import functools

import jax
import jax.numpy as jnp
from jax import lax
from jax.experimental import pallas as pl
from jax.experimental.pallas import tpu as pltpu
from jax.experimental.pallas import tpu_sc as plsc

VOCAB = 1000000
EMBED_DIM = 32
BATCH = 4096
HIST = 50

NC, NS = 2, 16
NW = NC * NS
TOTAL = BATCH * HIST
B_PER_W = TOTAL // NW
CHUNK = 1600
NCHUNK = B_PER_W // CHUNK

LANES = 128
VTILES = (VOCAB + LANES - 1) // LANES
VPAD = VTILES * LANES
TBL_WORDS = VPAD * EMBED_DIM

K = 2
SLOT_LANES = K * LANES
SLOT_WORDS = SLOT_LANES * EMBED_DIM
NGRP = (VTILES - 1) // K
NSLOT = 2 * ((NGRP + 2 * NW - 1) // (2 * NW))

_mesh = plsc.VectorSubcoreMesh(core_axis_name="c", subcore_axis_name="s")


@functools.partial(
    pl.kernel,
    mesh=_mesh,
    out_type=(
        jax.ShapeDtypeStruct((TBL_WORDS,), jnp.float32),
        jax.ShapeDtypeStruct((TOTAL,), jnp.int32),
    ),
    scratch_types=[
        pltpu.VMEM((32, SLOT_LANES), jnp.float32),
        pltpu.VMEM((32, SLOT_LANES), jnp.float32),
        pltpu.VMEM((SLOT_WORDS,), jnp.float32),
        pltpu.VMEM((SLOT_WORDS,), jnp.float32),
        pltpu.VMEM((8, LANES), jnp.int32),
        pltpu.VMEM((B_PER_W,), jnp.int32),
        pltpu.SemaphoreType.DMA,
        pltpu.SemaphoreType.DMA,
        pltpu.SemaphoreType.DMA,
        pltpu.SemaphoreType.DMA,
        pltpu.SemaphoreType.DMA,
    ],
    compiler_params=pltpu.CompilerParams(
        use_tc_tiling_on_sc=True, needs_layout_passes=False),
)
def _reformat_kernel(tab_hbm, idx_hbm, tbl_out, idx_out,
                     stg0, stg1, obuf0, obuf1, istg, iblk,
                     sin0, sin1, sout0, sout1, sem):
    wid = lax.axis_index("s") * NC + lax.axis_index("c")
    iota = lax.broadcasted_iota(jnp.int32, (16,), 0)

    ilane = iota * HIST
    for k in range(7):
        hstart = 8 * k
        nrows = min(8, HIST - hstart)
        pltpu.async_copy(
            idx_hbm.at[pl.ds(hstart, nrows), pl.ds(wid * LANES, LANES)],
            istg.at[pl.ds(0, nrows)], sem).wait()
        for r in range(nrows):
            h = hstart + r
            vs = [istg[r, pl.ds(g * 16, 16)] for g in range(8)]
            for g in range(8):
                plsc.store_scatter(iblk, [ilane + (g * 16 * HIST + h)], vs[g])
    pltpu.async_copy(iblk, idx_out.at[pl.ds(wid * B_PER_W, B_PER_W)],
                     sem).wait()

    vlane = iota * EMBED_DIM
    stg = (stg0, stg1)
    obuf = (obuf0, obuf1)
    sin = (sin0, sin1)
    sout = (sout0, sout1)

    def grp(s):
        return s * NW + wid

    def start_in(s, p):
        @pl.when(grp(s) < NGRP)
        def _():
            off = pl.multiple_of(grp(s) * SLOT_LANES, LANES)
            pltpu.make_async_copy(
                tab_hbm.at[:, pl.ds(off, SLOT_LANES)], stg[p], sin[p]).start()

    def wait_in(s, p):
        @pl.when(grp(s) < NGRP)
        def _():
            pltpu.make_async_copy(
                tab_hbm.at[:, pl.ds(0, SLOT_LANES)], stg[p], sin[p]).wait()

    def out_copy(s, p):
        off = pl.multiple_of(grp(s) * SLOT_WORDS, 8)
        return pltpu.make_async_copy(
            obuf[p], tbl_out.at[pl.ds(off, SLOT_WORDS)], sout[p])

    def start_out(s, p):
        @pl.when(grp(s) < NGRP)
        def _():
            out_copy(s, p).start()

    def wait_out(s, p):
        @pl.when(jnp.logical_and(s >= 0, grp(s) < NGRP))
        def _():
            out_copy(s, p).wait()

    def transpose_slot(p):
        src, dst = stg[p], obuf[p]
        for c in range(K):
            for e in range(EMBED_DIM):
                base = c * (LANES * EMBED_DIM) + e
                vs = [src[e, pl.ds(c * LANES + g * 16, 16)] for g in range(8)]
                for g in range(8):
                    plsc.store_scatter(
                        dst, [vlane + (base + g * 16 * EMBED_DIM)], vs[g])

    start_in(0, 0)

    def body(j, _):
        for p in range(2):
            s = 2 * j + p
            start_in(s + 1, 1 - p)
            wait_in(s, p)
            wait_out(s - 2, p)
            transpose_slot(p)
            start_out(s, p)
        return _

    lax.fori_loop(0, NSLOT // 2, body, None, unroll=False)
    wait_out(NSLOT - 2, 0)
    wait_out(NSLOT - 1, 1)

    @pl.when(wid == 0)
    def _():
        off = VTILES - 1
        tail = pl.multiple_of((off + wid) * LANES, LANES)
        pltpu.async_copy(
            tab_hbm.at[:, pl.ds(tail, LANES)],
            stg0.at[:, pl.ds(0, LANES)], sin0).wait()
        for e in range(EMBED_DIM):
            vs = [stg0[e, pl.ds(g * 16, 16)] for g in range(8)]
            for g in range(8):
                plsc.store_scatter(
                    obuf0, [vlane + (e + g * 16 * EMBED_DIM)], vs[g])
        pltpu.async_copy(
            obuf0.at[pl.ds(0, LANES * EMBED_DIM)],
            tbl_out.at[pl.ds(off * LANES * EMBED_DIM, LANES * EMBED_DIM)],
            sout0).wait()


@functools.partial(
    pl.kernel,
    mesh=_mesh,
    out_type=jax.ShapeDtypeStruct((BATCH, HIST, EMBED_DIM), jnp.float32),
    scratch_types=[
        pltpu.VMEM((CHUNK,), jnp.int32),
        pltpu.VMEM((CHUNK, EMBED_DIM), jnp.float32),
        pltpu.SemaphoreType.DMA,
    ],
    compiler_params=pltpu.CompilerParams(use_tc_tiling_on_sc=False),
)
def _gather_kernel(table_hbm, idx_hbm, out_hbm, idx_v, rows_v, sem):
    wid = lax.axis_index("s") * NC + lax.axis_index("c")
    base = wid * B_PER_W
    bstart = wid * (BATCH // NW)
    for c in range(NCHUNK):
        off = base + c * CHUNK
        pltpu.sync_copy(idx_hbm.at[pl.ds(off, CHUNK)], idx_v)
        pltpu.async_copy(table_hbm.at[idx_v], rows_v, sem).wait()
        for b in range(CHUNK // HIST):
            pltpu.sync_copy(rows_v.at[pl.ds(b * HIST, HIST)],
                            out_hbm.at[bstart + c * (CHUNK // HIST) + b])


def kernel(inputs, embeddings):
    tbl_lin, idx_lin = _reformat_kernel(embeddings.T, inputs.T)
    table = jnp.reshape(tbl_lin, (VPAD, EMBED_DIM))
    return _gather_kernel(table, idx_lin)
